# Initial kernel scaffold; baseline (speedup 1.0000x reference)
import jax
import jax.numpy as jnp
from jax.experimental import pallas as pl


def kernel(wav, offsets):
    raise NotImplementedError("write your pallas kernel here")


# speedup vs baseline: 5.4372x; 5.4372x over previous
import functools

import jax
import jax.numpy as jnp
from jax import lax
from jax.experimental import pallas as pl
from jax.experimental.pallas import tpu as pltpu
from jax.experimental.pallas import tpu_sc as plsc

SHIFT = 8192
ROWS = 64
L_IN = 160000
L_OUT = L_IN - SHIFT
N_CHUNK = 4
CH = L_OUT // N_CHUNK
ROWS_PER_W = 2


def _shift_body(w_hbm, offs_hbm, out_hbm, offs_v, buf_in, buf_out):
    nc = 2
    wid = lax.axis_index("s") * nc + lax.axis_index("c")
    pltpu.sync_copy(offs_hbm, offs_v)
    off_vec = offs_v[wid]
    for j in range(ROWS_PER_W):
        r = wid * ROWS_PER_W + j
        off = off_vec[j]
        a = off % 8
        base = pl.multiple_of(r * L_IN + off - a, 8)
        out_base = r * L_OUT
        for k in range(N_CHUNK):
            pltpu.sync_copy(w_hbm.at[pl.ds(base + k * CH, CH + 8)], buf_in)

            def shift(i, a=a):
                buf_out[pl.ds(i * 16, 16)] = buf_in[pl.ds(i * 16 + a, 16)]
                return i + 1

            lax.fori_loop(0, CH // 16, shift, 0, unroll=8)
            pltpu.sync_copy(
                buf_out,
                out_hbm.at[pl.ds(pl.multiple_of(out_base + k * CH, 8), CH)],
            )


@jax.jit
def kernel(wav, offsets):
    w = wav[0].reshape(ROWS * L_IN)
    offs = jnp.pad(offsets.reshape(32, 2), ((0, 0), (0, 14)))
    run = pl.kernel(
        _shift_body,
        out_type=jax.ShapeDtypeStruct((ROWS * L_OUT,), jnp.float32),
        mesh=plsc.VectorSubcoreMesh(core_axis_name="c", subcore_axis_name="s"),
        scratch_types=[
            pltpu.VMEM((32, 16), jnp.int32),
            pltpu.VMEM((CH + 8,), jnp.float32),
            pltpu.VMEM((CH,), jnp.float32),
        ],
    )
    out = run(w, offs)
    return out.reshape(2, 32, 1, L_OUT)
